# baseline (speedup 1.0000x reference)
import functools

import jax
import jax.numpy as jnp
from jax.experimental import pallas as pl
from jax.experimental.pallas import tpu as pltpu

N = 10000
M = 4096
D = 128
TM = 512
TH = 256


def _body(xnt_ref, h_ref, dv_ref, de_ref, o_ref, hbf_ref):
    jj = pl.program_id(0)

    @pl.when(jj == 0)
    def _init():
        o_ref[...] = jnp.zeros_like(o_ref)

    for t in range(TM // TH):
        sl = slice(t * TH, (t + 1) * TH)
        hbf_ref[...] = h_ref[:, sl].astype(jnp.bfloat16)
        xet = jax.lax.dot_general(
            xnt_ref[...], hbf_ref[...], (((1,), (0,)), ((), ())),
            preferred_element_type=jnp.float32)
        xet = (de_ref[:, sl] * xet).astype(jnp.bfloat16)
        o_ref[...] += jax.lax.dot_general(
            hbf_ref[...], xet, (((1,), (1,)), ((), ())),
            preferred_element_type=jnp.float32)

    @pl.when(jj == pl.num_programs(0) - 1)
    def _finish():
        o_ref[...] = dv_ref[...] * o_ref[...]


@functools.partial(jax.jit, static_argnames=())
def kernel(X, H, Dv_inv_sqrt, De_inv):
    xnt = (Dv_inv_sqrt[:, None] * X).astype(jnp.bfloat16).T
    dv = Dv_inv_sqrt.reshape(N, 1).astype(jnp.bfloat16)
    de = De_inv.reshape(1, M)
    grid = (M // TM,)
    return pl.pallas_call(
        _body,
        grid=grid,
        in_specs=[
            pl.BlockSpec((D, N), lambda jj: (0, 0)),
            pl.BlockSpec((N, TM), lambda jj: (0, jj)),
            pl.BlockSpec((N, 1), lambda jj: (0, 0)),
            pl.BlockSpec((1, TM), lambda jj: (0, jj)),
        ],
        out_specs=pl.BlockSpec((N, D), lambda jj: (0, 0)),
        out_shape=jax.ShapeDtypeStruct((N, D), jnp.float32),
        scratch_shapes=[pltpu.VMEM((N, TH), jnp.bfloat16)],
    )(xnt, H, dv, de)


# speedup vs baseline: 1.0090x; 1.0090x over previous
import functools

import jax
import jax.numpy as jnp
from jax.experimental import pallas as pl
from jax.experimental.pallas import tpu as pltpu

N = 10000
M = 4096
D = 128
TM = 512
TH = 256


def _body(xnt_ref, h_ref, dv_ref, de_ref, o_ref, hbuf_ref, hbf_ref, sem):
    jj = pl.program_id(0)
    nj = pl.num_programs(0)

    @pl.when(jj == 0)
    def _prologue():
        pltpu.make_async_copy(
            h_ref.at[:, pl.ds(0, TM)], hbuf_ref.at[0], sem.at[0]).start()
        o_ref[...] = jnp.zeros_like(o_ref)

    @pl.when(jj + 1 < nj)
    def _prefetch():
        pltpu.make_async_copy(
            h_ref.at[:, pl.ds((jj + 1) * TM, TM)],
            hbuf_ref.at[(jj + 1) % 2], sem.at[(jj + 1) % 2]).start()

    pltpu.make_async_copy(
        h_ref.at[:, pl.ds(jj * TM, TM)],
        hbuf_ref.at[jj % 2], sem.at[jj % 2]).wait()
    hw = hbuf_ref.at[jj % 2]

    for t in range(TM // TH):
        sl = slice(t * TH, (t + 1) * TH)
        hbf_ref[...] = hw[:, sl].astype(jnp.bfloat16)
        xet = jax.lax.dot_general(
            xnt_ref[...], hbf_ref[...], (((1,), (0,)), ((), ())),
            preferred_element_type=jnp.float32)
        xet = (de_ref[:, sl] * xet).astype(jnp.bfloat16)
        o_ref[...] += jax.lax.dot_general(
            hbf_ref[...], xet, (((1,), (1,)), ((), ())),
            preferred_element_type=jnp.float32)

    @pl.when(jj == nj - 1)
    def _finish():
        o_ref[...] = dv_ref[...] * o_ref[...]


@functools.partial(jax.jit, static_argnames=())
def kernel(X, H, Dv_inv_sqrt, De_inv):
    xnt = (Dv_inv_sqrt[:, None] * X).astype(jnp.bfloat16).T
    dv = Dv_inv_sqrt.reshape(N, 1).astype(jnp.bfloat16)
    de = De_inv.reshape(1, M)
    grid = (M // TM,)
    return pl.pallas_call(
        _body,
        grid=grid,
        in_specs=[
            pl.BlockSpec((D, N), lambda jj: (0, 0)),
            pl.BlockSpec(memory_space=pltpu.MemorySpace.HBM),
            pl.BlockSpec((N, 1), lambda jj: (0, 0)),
            pl.BlockSpec((1, TM), lambda jj: (0, jj)),
        ],
        out_specs=pl.BlockSpec((N, D), lambda jj: (0, 0)),
        out_shape=jax.ShapeDtypeStruct((N, D), jnp.float32),
        scratch_shapes=[
            pltpu.VMEM((2, N, TM), jnp.float32),
            pltpu.VMEM((N, TH), jnp.bfloat16),
            pltpu.SemaphoreType.DMA((2,)),
        ],
    )(xnt, H, dv, de)


# speedup vs baseline: 1.0898x; 1.0801x over previous
import functools

import jax
import jax.numpy as jnp
from jax.experimental import pallas as pl
from jax.experimental.pallas import tpu as pltpu

N = 10000
M = 4096
D = 128
TM = 512
TH = 512


def _body(xnt_ref, h_ref, dv_ref, de_ref, o_ref, hbuf_ref, sem):
    jj = pl.program_id(0)
    nj = pl.num_programs(0)

    @pl.when(jj == 0)
    def _prologue():
        pltpu.make_async_copy(
            h_ref.at[:, pl.ds(0, TM)], hbuf_ref.at[0], sem.at[0]).start()
        o_ref[...] = jnp.zeros_like(o_ref)

    @pl.when(jj + 1 < nj)
    def _prefetch():
        pltpu.make_async_copy(
            h_ref.at[:, pl.ds((jj + 1) * TM, TM)],
            hbuf_ref.at[(jj + 1) % 2], sem.at[(jj + 1) % 2]).start()

    pltpu.make_async_copy(
        h_ref.at[:, pl.ds(jj * TM, TM)],
        hbuf_ref.at[jj % 2], sem.at[jj % 2]).wait()
    hw = hbuf_ref.at[jj % 2]

    for t in range(TM // TH):
        sl = slice(t * TH, (t + 1) * TH)
        hh = hw[:, sl]
        xet = jax.lax.dot_general(
            xnt_ref[...], hh, (((1,), (0,)), ((), ())),
            preferred_element_type=jnp.float32,
            precision=jax.lax.Precision.DEFAULT)
        xet = de_ref[:, sl] * xet
        o_ref[...] += jax.lax.dot_general(
            hh, xet, (((1,), (1,)), ((), ())),
            preferred_element_type=jnp.float32,
            precision=jax.lax.Precision.DEFAULT)

    @pl.when(jj == nj - 1)
    def _finish():
        o_ref[...] = dv_ref[...] * o_ref[...]


@functools.partial(jax.jit, static_argnames=())
def kernel(X, H, Dv_inv_sqrt, De_inv):
    xnt = (Dv_inv_sqrt[:, None] * X).T
    dv = Dv_inv_sqrt.reshape(N, 1).astype(jnp.bfloat16)
    de = De_inv.reshape(1, M)
    grid = (M // TM,)
    return pl.pallas_call(
        _body,
        grid=grid,
        in_specs=[
            pl.BlockSpec((D, N), lambda jj: (0, 0)),
            pl.BlockSpec(memory_space=pltpu.MemorySpace.HBM),
            pl.BlockSpec((N, 1), lambda jj: (0, 0)),
            pl.BlockSpec((1, TM), lambda jj: (0, jj)),
        ],
        out_specs=pl.BlockSpec((N, D), lambda jj: (0, 0)),
        out_shape=jax.ShapeDtypeStruct((N, D), jnp.float32),
        scratch_shapes=[
            pltpu.VMEM((2, N, TM), jnp.float32),
            pltpu.SemaphoreType.DMA((2,)),
        ],
    )(xnt, H, dv, de)


# speedup vs baseline: 1.0906x; 1.0007x over previous
import functools

import jax
import jax.numpy as jnp
from jax.experimental import pallas as pl
from jax.experimental.pallas import tpu as pltpu

N = 10000
M = 4096
D = 128
TM = 512
HH = N // 2
TH = 512


def _body(xnt_ref, h_ref, dv_ref, de_ref, o_ref, hbuf_ref, sem):
    jj = pl.program_id(0)
    nj = pl.num_programs(0)

    def _copies(idx, buf):
        return [
            pltpu.make_async_copy(
                h_ref.at[pl.ds(h * HH, HH), pl.ds(idx * TM, TM)],
                hbuf_ref.at[buf, pl.ds(h * HH, HH)], sem.at[buf, h])
            for h in range(2)
        ]

    @pl.when(jj == 0)
    def _prologue():
        for c in _copies(0, 0):
            c.start()
        o_ref[...] = jnp.zeros_like(o_ref)

    @pl.when(jj + 1 < nj)
    def _prefetch():
        for c in _copies(jj + 1, (jj + 1) % 2):
            c.start()

    for c in _copies(jj, jj % 2):
        c.wait()
    hw = hbuf_ref.at[jj % 2]

    for t in range(TM // TH):
        sl = slice(t * TH, (t + 1) * TH)
        hh = hw[:, sl]
        xet = jax.lax.dot_general(
            xnt_ref[...], hh, (((1,), (0,)), ((), ())),
            preferred_element_type=jnp.float32,
            precision=jax.lax.Precision.DEFAULT)
        xet = de_ref[:, sl] * xet
        o_ref[...] += jax.lax.dot_general(
            hh, xet, (((1,), (1,)), ((), ())),
            preferred_element_type=jnp.float32,
            precision=jax.lax.Precision.DEFAULT)

    @pl.when(jj == nj - 1)
    def _finish():
        o_ref[...] = dv_ref[...] * o_ref[...]


@functools.partial(jax.jit, static_argnames=())
def kernel(X, H, Dv_inv_sqrt, De_inv):
    xnt = (Dv_inv_sqrt[:, None] * X).T
    dv = Dv_inv_sqrt.reshape(N, 1).astype(jnp.bfloat16)
    de = De_inv.reshape(1, M)
    grid = (M // TM,)
    return pl.pallas_call(
        _body,
        grid=grid,
        in_specs=[
            pl.BlockSpec((D, N), lambda jj: (0, 0)),
            pl.BlockSpec(memory_space=pltpu.MemorySpace.HBM),
            pl.BlockSpec((N, 1), lambda jj: (0, 0)),
            pl.BlockSpec((1, TM), lambda jj: (0, jj)),
        ],
        out_specs=pl.BlockSpec((N, D), lambda jj: (0, 0)),
        out_shape=jax.ShapeDtypeStruct((N, D), jnp.float32),
        scratch_shapes=[
            pltpu.VMEM((2, N, TM), jnp.float32),
            pltpu.SemaphoreType.DMA((2, 2)),
        ],
    )(xnt, H, dv, de)


# speedup vs baseline: 1.1020x; 1.0105x over previous
import functools

import jax
import jax.numpy as jnp
from jax.experimental import pallas as pl
from jax.experimental.pallas import tpu as pltpu

N = 10000
M = 4096
D = 128
TM = 512
HH = N // 2
TH = 512


def _body(xnt_ref, h_ref, dv_ref, de_ref, o_ref):
    jj = pl.program_id(0)
    nj = pl.num_programs(0)

    @pl.when(jj == 0)
    def _prologue():
        o_ref[...] = jnp.zeros_like(o_ref)

    hw = h_ref

    for t in range(TM // TH):
        sl = slice(t * TH, (t + 1) * TH)
        hh = hw[:, sl]
        xet = jax.lax.dot_general(
            xnt_ref[...], hh, (((1,), (0,)), ((), ())),
            preferred_element_type=jnp.float32,
            precision=jax.lax.Precision.DEFAULT)
        xet = de_ref[:, sl] * xet
        o_ref[...] += jax.lax.dot_general(
            hh, xet, (((1,), (1,)), ((), ())),
            preferred_element_type=jnp.float32,
            precision=jax.lax.Precision.DEFAULT)

    @pl.when(jj == nj - 1)
    def _finish():
        o_ref[...] = dv_ref[...] * o_ref[...]


@functools.partial(jax.jit, static_argnames=())
def kernel(X, H, Dv_inv_sqrt, De_inv):
    xnt = (Dv_inv_sqrt[:, None] * X).T
    dv = Dv_inv_sqrt.reshape(N, 1).astype(jnp.bfloat16)
    de = De_inv.reshape(1, M)
    grid = (M // TM,)
    return pl.pallas_call(
        _body,
        grid=grid,
        in_specs=[
            pl.BlockSpec((D, N), lambda jj: (0, 0)),
            pl.BlockSpec((N, TM), lambda jj: (0, jj)),
            pl.BlockSpec((N, 1), lambda jj: (0, 0)),
            pl.BlockSpec((1, TM), lambda jj: (0, jj)),
        ],
        out_specs=pl.BlockSpec((N, D), lambda jj: (0, 0)),
        out_shape=jax.ShapeDtypeStruct((N, D), jnp.float32),
    )(xnt, H, dv, de)
